# baseline (speedup 1.0000x reference)
import functools
import math

import jax
import jax.numpy as jnp
from jax.experimental import pallas as pl
from jax.experimental.pallas import tpu as pltpu

_C = 10
_B = 2048
_EPS = 0.05
_ROWS = 256
_K0 = 1.0 + math.exp(2.0 * _EPS)


def _softmax(x):
    m = jnp.max(x, axis=1, keepdims=True)
    e = jnp.exp(x - m)
    return e / jnp.sum(e, axis=1, keepdims=True)


def _auc_kernel(ys_ref, ysa_ref, labc_ref, labr_ref, emp_ref, disc_ref,
                e4p_ref, e4pi_ref, t2_ref, t2i_ref):
    i = pl.program_id(0)

    @pl.when(i == 0)
    def _build_tables():
        p = _softmax(ys_ref[...])
        pa = _softmax(ysa_ref[...])
        e4p = jnp.exp(4.0 * p)
        t2 = jnp.exp(2.0 * (p - pa))
        e4p_ref[...] = e4p
        e4pi_ref[...] = 1.0 / e4p
        t2_ref[...] = t2
        t2i_ref[...] = 1.0 / t2
        emp_ref[...] = jnp.zeros((1, 1), jnp.float32)
        disc_ref[...] = jnp.zeros((1, 1), jnp.float32)

    lab_col = labc_ref[...]
    lab_row = labr_ref[...]

    cls = jax.lax.broadcasted_iota(jnp.int32, (1, _C), 1)
    onehot = (lab_col == cls).astype(jnp.float32)

    rows = pl.ds(i * _ROWS, _ROWS)
    e4g = jnp.sum(onehot * e4p_ref[rows, :], axis=1, keepdims=True)
    t2g = jnp.sum(onehot * t2_ref[rows, :], axis=1, keepdims=True)
    c_e = math.exp(_EPS + 4.0) / e4g
    c_e_inv = math.exp(_EPS - 4.0) * e4g
    c_s = math.exp(_EPS) / t2g
    c_s_inv = math.exp(_EPS) * t2g

    dot = functools.partial(
        jax.lax.dot_general,
        dimension_numbers=(((1,), (1,)), ((), ())),
        preferred_element_type=jnp.float32,
        precision=jax.lax.Precision.HIGHEST,
    )
    h_e = dot(onehot * c_e, e4p_ref[...])
    r_e = dot(onehot * c_e_inv, e4pi_ref[...])
    h_s = dot(onehot * c_s, t2_ref[...])
    r_s = dot(onehot * c_s_inv, t2i_ref[...])

    w = jnp.zeros_like(c_e)
    for c in range(_C):
        n_c = jnp.sum((lab_row == c).astype(jnp.float32))
        fac_c = math.log(2.0) / (n_c * (_B - n_c))
        w = w + jnp.where(lab_col == c, fac_c, 0.0)
    wv = jnp.where(lab_col != lab_row, w, 0.0)

    l_e = jnp.log2(_K0 + h_e + r_e)
    l_s = jnp.log2(_K0 + h_s + r_s)
    emp = jnp.sum(wv * l_e).reshape(1, 1)
    disc = jnp.sum(wv * l_s).reshape(1, 1)

    emp_ref[...] += emp
    disc_ref[...] += disc


def kernel(y_s, y_s_adv, labels_s, y_t, y_t_adv, epoch):
    lab = labels_s.astype(jnp.int32)
    lab_col = lab.reshape(_B, 1)
    lab_row = lab.reshape(1, _B)

    grid = (_B // _ROWS,)
    emp, disc = pl.pallas_call(
        _auc_kernel,
        grid=grid,
        in_specs=[
            pl.BlockSpec((_B, _C), lambda i: (0, 0)),
            pl.BlockSpec((_B, _C), lambda i: (0, 0)),
            pl.BlockSpec((_ROWS, 1), lambda i: (i, 0)),
            pl.BlockSpec((1, _B), lambda i: (0, 0)),
        ],
        out_specs=[
            pl.BlockSpec((1, 1), lambda i: (0, 0)),
            pl.BlockSpec((1, 1), lambda i: (0, 0)),
        ],
        out_shape=[
            jax.ShapeDtypeStruct((1, 1), jnp.float32),
            jax.ShapeDtypeStruct((1, 1), jnp.float32),
        ],
        scratch_shapes=[
            pltpu.VMEM((_B, _C), jnp.float32),
            pltpu.VMEM((_B, _C), jnp.float32),
            pltpu.VMEM((_B, _C), jnp.float32),
            pltpu.VMEM((_B, _C), jnp.float32),
        ],
    )(y_s, y_s_adv, lab_col, lab_row)

    empirical = 0.25 * emp[0, 0]
    transfer = -0.5 * disc[0, 0]
    return (empirical, transfer)


# speedup vs baseline: 2.3088x; 2.3088x over previous
import functools
import math

import jax
import jax.numpy as jnp
from jax.experimental import pallas as pl
from jax.experimental.pallas import tpu as pltpu

_C = 10
_B = 2048
_EPS = 0.05
_ROWS = 256
_K0 = 1.0 + math.exp(2.0 * _EPS)


def _softmax(x):
    m = jnp.max(x, axis=1, keepdims=True)
    e = jnp.exp(x - m)
    return e / jnp.sum(e, axis=1, keepdims=True)


def _auc_kernel(ys_ref, ysa_ref, labc_ref, labr_ref, emp_ref, disc_ref,
                e4p_ref, e4pi_ref, t2_ref, t2i_ref):
    i = pl.program_id(0)

    @pl.when(i == 0)
    def _build_tables():
        p = _softmax(ys_ref[...])
        pa = _softmax(ysa_ref[...])
        e4p = jnp.exp(4.0 * p)
        t2 = jnp.exp(2.0 * (p - pa))
        e4p_ref[...] = e4p
        e4pi_ref[...] = 1.0 / e4p
        t2_ref[...] = t2
        t2i_ref[...] = 1.0 / t2
        emp_ref[...] = jnp.zeros((1, 1), jnp.float32)
        disc_ref[...] = jnp.zeros((1, 1), jnp.float32)

    lab_col = labc_ref[...]
    lab_row = labr_ref[...]

    cls = jax.lax.broadcasted_iota(jnp.int32, (1, _C), 1)
    onehot = (lab_col == cls).astype(jnp.float32)

    rows = pl.ds(i * _ROWS, _ROWS)
    e4g = jnp.sum(onehot * e4p_ref[rows, :], axis=1, keepdims=True)
    t2g = jnp.sum(onehot * t2_ref[rows, :], axis=1, keepdims=True)
    c_e = math.exp(_EPS + 4.0) / e4g
    c_e_inv = math.exp(_EPS - 4.0) * e4g
    c_s = math.exp(_EPS) / t2g
    c_s_inv = math.exp(_EPS) * t2g

    dot = functools.partial(
        jax.lax.dot_general,
        dimension_numbers=(((1,), (1,)), ((), ())),
        preferred_element_type=jnp.float32,
        precision=jax.lax.Precision.DEFAULT,
    )
    h_e = dot(onehot * c_e, e4p_ref[...])
    r_e = dot(onehot * c_e_inv, e4pi_ref[...])
    h_s = dot(onehot * c_s, t2_ref[...])
    r_s = dot(onehot * c_s_inv, t2i_ref[...])

    w = jnp.zeros_like(c_e)
    for c in range(_C):
        n_c = jnp.sum((lab_row == c).astype(jnp.float32))
        fac_c = math.log(2.0) / (n_c * (_B - n_c))
        w = w + jnp.where(lab_col == c, fac_c, 0.0)
    wv = jnp.where(lab_col != lab_row, w, 0.0)

    l_e = jnp.log2(_K0 + h_e + r_e)
    l_s = jnp.log2(_K0 + h_s + r_s)
    emp = jnp.sum(wv * l_e).reshape(1, 1)
    disc = jnp.sum(wv * l_s).reshape(1, 1)

    emp_ref[...] += emp
    disc_ref[...] += disc


def kernel(y_s, y_s_adv, labels_s, y_t, y_t_adv, epoch):
    lab = labels_s.astype(jnp.int32)
    lab_col = lab.reshape(_B, 1)
    lab_row = lab.reshape(1, _B)

    grid = (_B // _ROWS,)
    emp, disc = pl.pallas_call(
        _auc_kernel,
        grid=grid,
        in_specs=[
            pl.BlockSpec((_B, _C), lambda i: (0, 0)),
            pl.BlockSpec((_B, _C), lambda i: (0, 0)),
            pl.BlockSpec((_ROWS, 1), lambda i: (i, 0)),
            pl.BlockSpec((1, _B), lambda i: (0, 0)),
        ],
        out_specs=[
            pl.BlockSpec((1, 1), lambda i: (0, 0)),
            pl.BlockSpec((1, 1), lambda i: (0, 0)),
        ],
        out_shape=[
            jax.ShapeDtypeStruct((1, 1), jnp.float32),
            jax.ShapeDtypeStruct((1, 1), jnp.float32),
        ],
        scratch_shapes=[
            pltpu.VMEM((_B, _C), jnp.float32),
            pltpu.VMEM((_B, _C), jnp.float32),
            pltpu.VMEM((_B, _C), jnp.float32),
            pltpu.VMEM((_B, _C), jnp.float32),
        ],
    )(y_s, y_s_adv, lab_col, lab_row)

    empirical = 0.25 * emp[0, 0]
    transfer = -0.5 * disc[0, 0]
    return (empirical, transfer)


# speedup vs baseline: 2.3297x; 1.0090x over previous
import functools
import math

import jax
import jax.numpy as jnp
from jax.experimental import pallas as pl
from jax.experimental.pallas import tpu as pltpu

_C = 10
_B = 2048
_EPS = 0.05
_ROWS = 256
_K0 = 1.0 + math.exp(2.0 * _EPS)


def _softmax(x):
    m = jnp.max(x, axis=1, keepdims=True)
    e = jnp.exp(x - m)
    return e / jnp.sum(e, axis=1, keepdims=True)


def _auc_kernel(ys_ref, ysa_ref, labc_ref, labr_ref, emp_ref, disc_ref,
                e4p_ref, e4pi_ref, t2_ref, t2i_ref, w_ref, acc_e_ref,
                acc_s_ref):
    i = pl.program_id(0)
    nsteps = pl.num_programs(0)
    lab_row = labr_ref[...]

    @pl.when(i == 0)
    def _build_tables():
        p = _softmax(ys_ref[...])
        pa = _softmax(ysa_ref[...])
        e4p = jnp.exp(4.0 * p)
        t2 = jnp.exp(2.0 * (p - pa))
        e4p_ref[...] = e4p
        e4pi_ref[...] = 1.0 / e4p
        t2_ref[...] = t2
        t2i_ref[...] = 1.0 / t2
        lab_all = labc_ref[...]
        w = jnp.zeros((_B, 1), jnp.float32)
        for c in range(_C):
            n_c = jnp.sum((lab_row == c).astype(jnp.float32))
            fac_c = math.log(2.0) / (n_c * (_B - n_c))
            w = w + jnp.where(lab_all == c, fac_c, 0.0)
        w_ref[...] = w
        acc_e_ref[...] = jnp.zeros((1, _B), jnp.float32)
        acc_s_ref[...] = jnp.zeros((1, _B), jnp.float32)

    rows = pl.ds(i * _ROWS, _ROWS)
    lab_blk = labc_ref[rows, :]

    cls = jax.lax.broadcasted_iota(jnp.int32, (1, _C), 1)
    onehot = (lab_blk == cls).astype(jnp.float32)

    e4g = jnp.sum(onehot * e4p_ref[rows, :], axis=1, keepdims=True)
    t2g = jnp.sum(onehot * t2_ref[rows, :], axis=1, keepdims=True)
    c_e = math.exp(_EPS + 4.0) / e4g
    c_e_inv = math.exp(_EPS - 4.0) * e4g
    c_s = math.exp(_EPS) / t2g
    c_s_inv = math.exp(_EPS) * t2g

    dot = functools.partial(
        jax.lax.dot_general,
        dimension_numbers=(((1,), (1,)), ((), ())),
        preferred_element_type=jnp.float32,
        precision=jax.lax.Precision.DEFAULT,
    )
    h_e = dot(onehot * c_e, e4p_ref[...])
    r_e = dot(onehot * c_e_inv, e4pi_ref[...])
    h_s = dot(onehot * c_s, t2_ref[...])
    r_s = dot(onehot * c_s_inv, t2i_ref[...])

    wv = jnp.where(lab_blk != lab_row, w_ref[rows, :], 0.0)

    l_e = jnp.log2(_K0 + h_e + r_e)
    l_s = jnp.log2(_K0 + h_s + r_s)
    acc_e_ref[...] += jnp.sum(wv * l_e, axis=0, keepdims=True)
    acc_s_ref[...] += jnp.sum(wv * l_s, axis=0, keepdims=True)

    @pl.when(i == nsteps - 1)
    def _finish():
        emp_ref[...] = jnp.sum(acc_e_ref[...]).reshape(1, 1)
        disc_ref[...] = jnp.sum(acc_s_ref[...]).reshape(1, 1)


def kernel(y_s, y_s_adv, labels_s, y_t, y_t_adv, epoch):
    lab = labels_s.astype(jnp.int32)
    lab_col = lab.reshape(_B, 1)
    lab_row = lab.reshape(1, _B)

    grid = (_B // _ROWS,)
    emp, disc = pl.pallas_call(
        _auc_kernel,
        grid=grid,
        in_specs=[
            pl.BlockSpec((_B, _C), lambda i: (0, 0)),
            pl.BlockSpec((_B, _C), lambda i: (0, 0)),
            pl.BlockSpec((_B, 1), lambda i: (0, 0)),
            pl.BlockSpec((1, _B), lambda i: (0, 0)),
        ],
        out_specs=[
            pl.BlockSpec((1, 1), lambda i: (0, 0)),
            pl.BlockSpec((1, 1), lambda i: (0, 0)),
        ],
        out_shape=[
            jax.ShapeDtypeStruct((1, 1), jnp.float32),
            jax.ShapeDtypeStruct((1, 1), jnp.float32),
        ],
        scratch_shapes=[
            pltpu.VMEM((_B, _C), jnp.float32),
            pltpu.VMEM((_B, _C), jnp.float32),
            pltpu.VMEM((_B, _C), jnp.float32),
            pltpu.VMEM((_B, _C), jnp.float32),
            pltpu.VMEM((_B, 1), jnp.float32),
            pltpu.VMEM((1, _B), jnp.float32),
            pltpu.VMEM((1, _B), jnp.float32),
        ],
    )(y_s, y_s_adv, lab_col, lab_row)

    empirical = 0.25 * emp[0, 0]
    transfer = -0.5 * disc[0, 0]
    return (empirical, transfer)
